# Initial kernel scaffold; baseline (speedup 1.0000x reference)
import functools
import math

import jax
import jax.numpy as jnp
from jax import lax
from jax.experimental import pallas as pl
from jax.experimental.pallas import tpu as pltpu
from jax.experimental.pallas import tpu_sc as plsc

N = 10000
E = 320000
F_IN = 128
DK = 128
SCALE = 1.0 / math.sqrt(DK)

NC = 2
NS = 16
NW = NC * NS
EPW = E // NW
CH = 80
NCHUNK = EPW // CH
NPAD = 10240
SLICE = NPAD // NS

_f32 = jnp.float32
_i32 = jnp.int32

_MESH = plsc.VectorSubcoreMesh(core_axis_name="c", subcore_axis_name="s")


def _qkv_body(x_ref, wq_ref, wk_ref, wv_ref, q_ref, k_ref, v_ref):
    xb = x_ref[...]
    dn = (((1,), (1,)), ((), ()))
    q_ref[...] = lax.dot_general(xb, wq_ref[...], dn, preferred_element_type=_f32)
    k_ref[...] = lax.dot_general(xb, wk_ref[...], dn, preferred_element_type=_f32)
    v_ref[...] = lax.dot_general(xb, wv_ref[...], dn, preferred_element_type=_f32)


_BN = 400


def _qkv(x, wq, wk, wv):
    grid = (N // _BN,)
    return pl.pallas_call(
        _qkv_body,
        grid=grid,
        in_specs=[
            pl.BlockSpec((_BN, F_IN), lambda i: (i, 0)),
            pl.BlockSpec((DK, F_IN), lambda i: (0, 0)),
            pl.BlockSpec((DK, F_IN), lambda i: (0, 0)),
            pl.BlockSpec((DK, F_IN), lambda i: (0, 0)),
        ],
        out_specs=[
            pl.BlockSpec((_BN, DK), lambda i: (i, 0)),
            pl.BlockSpec((_BN, DK), lambda i: (i, 0)),
            pl.BlockSpec((_BN, DK), lambda i: (i, 0)),
        ],
        out_shape=[
            jax.ShapeDtypeStruct((N, DK), _f32),
            jax.ShapeDtypeStruct((N, DK), _f32),
            jax.ShapeDtypeStruct((N, DK), _f32),
        ],
    )(x, wq, wk, wv)


def _scores_body(k_hbm, q_hbm, row_hbm, col_hbm, d_hbm,
                 scores_hbm, tmax_hbm,
                 idx_r, idx_c, krows, qrows, dvec, sbuf, mxbuf, sem):
    cid = lax.axis_index("c")
    sid = lax.axis_index("s")
    wid = sid * NC + cid
    base0 = wid * EPW
    pltpu.sync_copy(d_hbm, dvec)
    lanes = lax.iota(_i32, 16)

    def chunk_body(i, mrun):
        base = base0 + i * CH
        pltpu.sync_copy(row_hbm.at[pl.ds(base, CH)], idx_r)
        pltpu.sync_copy(col_hbm.at[pl.ds(base, CH)], idx_c)
        pltpu.async_copy(k_hbm.at[idx_r], krows, sem).wait()
        pltpu.async_copy(q_hbm.at[idx_c], qrows, sem).wait()

        def grp(g, mr):
            ev = g * 16 + lanes

            def d_body(dd, accs):
                acc_e, acc_s = accs
                dl = jnp.full((16,), dd, _i32)
                dv = jnp.full((16,), dvec[dd])
                kv = plsc.load_gather(krows, [ev, dl])
                qv = plsc.load_gather(qrows, [ev, dl])
                diff = kv - qv
                return (acc_e + diff * diff * dv, acc_s + kv * qv * dv)

            acc_e, acc_s = lax.fori_loop(
                0, DK, d_body,
                (jnp.zeros((16,), _f32), jnp.zeros((16,), _f32)),
                unroll=2)
            rv = idx_r[pl.ds(g * 16, 16)]
            cv = idx_c[pl.ds(g * 16, 16)]
            sv = jnp.where(rv == cv, acc_s, acc_e) * SCALE
            sv = jnp.where(sv >= 0.0, sv, 0.2 * sv)
            sbuf[pl.ds(g * 16, 16)] = sv
            return jnp.maximum(mr, sv)

        mrun = lax.fori_loop(0, CH // 16, grp, mrun)
        pltpu.sync_copy(sbuf, scores_hbm.at[pl.ds(base, CH)])
        return mrun

    mrun = lax.fori_loop(0, NCHUNK, chunk_body,
                         jnp.full((16,), -jnp.inf, _f32))
    mxbuf[...] = mrun
    pltpu.sync_copy(mxbuf, tmax_hbm.at[wid])


def _scores(k, q, row, col, d):
    fn = functools.partial(
        pl.kernel,
        out_type=[
            jax.ShapeDtypeStruct((E,), _f32),
            jax.ShapeDtypeStruct((NW, 16), _f32),
        ],
        mesh=_MESH,
        scratch_types=[
            pltpu.VMEM((CH,), _i32),
            pltpu.VMEM((CH,), _i32),
            pltpu.VMEM((CH, DK), _f32),
            pltpu.VMEM((CH, DK), _f32),
            pltpu.VMEM((DK,), _f32),
            pltpu.VMEM((CH,), _f32),
            pltpu.VMEM((16,), _f32),
            pltpu.SemaphoreType.DMA,
        ],
    )(_scores_body)
    return fn(k, q, row, col, d)


def _expsum_body(scores_hbm, row_hbm, tmax_hbm,
                 ex_hbm, ssum2_hbm,
                 sbuf, rbuf, exbuf, ssum_v, tmv, tmp, accb, spart, sem):
    cid = lax.axis_index("c")
    sid = lax.axis_index("s")
    wid = sid * NC + cid
    base0 = wid * EPW
    pltpu.sync_copy(tmax_hbm, tmv)
    mv = jnp.full((16,), -jnp.inf, _f32)
    for i in range(NW):
        mv = jnp.maximum(mv, tmv[i, :])
    gmax = jnp.max(mv)

    def zb(i, _):
        ssum_v[pl.ds(i * 16, 16)] = jnp.zeros((16,), _f32)
        return 0

    lax.fori_loop(0, NPAD // 16, zb, 0)

    def chunk_body(i, _):
        base = base0 + i * CH
        pltpu.sync_copy(scores_hbm.at[pl.ds(base, CH)], sbuf)
        pltpu.sync_copy(row_hbm.at[pl.ds(base, CH)], rbuf)
        for g in range(CH // 16):
            sv = sbuf[pl.ds(g * 16, 16)]
            exbuf[pl.ds(g * 16, 16)] = jnp.exp(sv - gmax)
        pltpu.sync_copy(exbuf, ex_hbm.at[pl.ds(base, CH)])

        def eb(e, _):
            r = rbuf[e]
            ssum_v[r] = ssum_v[r] + exbuf[e]
            return 0

        lax.fori_loop(0, CH, eb, 0)
        return 0

    lax.fori_loop(0, NCHUNK, chunk_body, 0)

    pltpu.sync_copy(ssum_v, spart.at[sid])
    plsc.subcore_barrier()
    off = sid * SLICE
    pltpu.sync_copy(spart.at[0, pl.ds(off, SLICE)], accb)
    for p in range(1, NS):
        pltpu.sync_copy(spart.at[p, pl.ds(off, SLICE)], tmp)

        def ab(j, _):
            accb[pl.ds(j * 16, 16)] = (accb[pl.ds(j * 16, 16)]
                                       + tmp[pl.ds(j * 16, 16)])
            return 0

        lax.fori_loop(0, SLICE // 16, ab, 0)
    pltpu.sync_copy(accb, ssum2_hbm.at[cid, pl.ds(off, SLICE)])


def _expsum(scores, row, tmax):
    fn = functools.partial(
        pl.kernel,
        out_type=[
            jax.ShapeDtypeStruct((E,), _f32),
            jax.ShapeDtypeStruct((NC, NPAD), _f32),
        ],
        mesh=_MESH,
        scratch_types=[
            pltpu.VMEM((CH,), _f32),
            pltpu.VMEM((CH,), _i32),
            pltpu.VMEM((CH,), _f32),
            pltpu.VMEM((NPAD,), _f32),
            pltpu.VMEM((NW, 16), _f32),
            pltpu.VMEM((SLICE,), _f32),
            pltpu.VMEM((SLICE,), _f32),
            pltpu.VMEM_SHARED((NS, NPAD), _f32),
            pltpu.SemaphoreType.DMA,
        ],
    )(_expsum_body)
    return fn(scores, row, tmax)


def _scatter_body(ex_hbm, row_hbm, col_hbm, v_hbm, ssum2_hbm,
                  outp_hbm,
                  idx_r, idx_c, exbuf, abuf, vrows, msg, ssum_v, s2buf,
                  out_acc, sem):
    cid = lax.axis_index("c")
    sid = lax.axis_index("s")
    wid = sid * NC + cid
    base0 = wid * EPW

    pltpu.sync_copy(ssum2_hbm.at[0], ssum_v)
    pltpu.sync_copy(ssum2_hbm.at[1], s2buf)

    def ad(j, _):
        ssum_v[pl.ds(j * 16, 16)] = (ssum_v[pl.ds(j * 16, 16)]
                                     + s2buf[pl.ds(j * 16, 16)])
        return 0

    lax.fori_loop(0, NPAD // 16, ad, 0)

    def zm(j, _):
        msg[j // (DK // 16), pl.ds((j % (DK // 16)) * 16, 16)] = jnp.zeros((16,), _f32)
        return 0

    lax.fori_loop(0, CH * DK // 16, zm, 0)
    for h in range(SLICE // CH):
        pltpu.sync_copy(msg, out_acc.at[pl.ds(sid * SLICE + h * CH, CH)])
    plsc.subcore_barrier()

    def chunk_body(i, _):
        base = base0 + i * CH
        pltpu.sync_copy(row_hbm.at[pl.ds(base, CH)], idx_r)
        pltpu.sync_copy(col_hbm.at[pl.ds(base, CH)], idx_c)
        pltpu.sync_copy(ex_hbm.at[pl.ds(base, CH)], exbuf)
        pltpu.async_copy(v_hbm.at[idx_c], vrows, sem).wait()
        for g in range(CH // 16):
            rv = idx_r[pl.ds(g * 16, 16)]
            sden = plsc.load_gather(ssum_v, [rv])
            abuf[pl.ds(g * 16, 16)] = (exbuf[pl.ds(g * 16, 16)]
                                       / (sden + 1e-16))

        def eb(e, _):
            av = jnp.full((16,), abuf[e])
            for j in range(DK // 16):
                msg[e, pl.ds(j * 16, 16)] = vrows[e, pl.ds(j * 16, 16)] * av
            return 0

        lax.fori_loop(0, CH, eb, 0)
        pltpu.sync_copy(msg, out_acc.at[idx_r], add=True)
        return 0

    lax.fori_loop(0, NCHUNK, chunk_body, 0)
    plsc.subcore_barrier()
    pltpu.sync_copy(out_acc.at[pl.ds(sid * SLICE, SLICE)],
                    outp_hbm.at[cid, pl.ds(sid * SLICE, SLICE)])


def _scatter(ex, row, col, v, ssum2):
    fn = functools.partial(
        pl.kernel,
        out_type=[jax.ShapeDtypeStruct((NC, NPAD, DK), _f32)],
        mesh=_MESH,
        scratch_types=[
            pltpu.VMEM((CH,), _i32),
            pltpu.VMEM((CH,), _i32),
            pltpu.VMEM((CH,), _f32),
            pltpu.VMEM((CH,), _f32),
            pltpu.VMEM((CH, DK), _f32),
            pltpu.VMEM((CH, DK), _f32),
            pltpu.VMEM((NPAD,), _f32),
            pltpu.VMEM((NPAD,), _f32),
            pltpu.VMEM_SHARED((NPAD, DK), _f32),
            pltpu.SemaphoreType.DMA,
        ],
    )(_scatter_body)
    return fn(ex, row, col, v, ssum2)


def _add_body(p_ref, o_ref):
    o_ref[...] = p_ref[0] + p_ref[1]


def _final_add(outp):
    grid = (N // _BN,)
    return pl.pallas_call(
        _add_body,
        grid=grid,
        in_specs=[pl.BlockSpec((NC, _BN, DK), lambda i: (0, i, 0))],
        out_specs=pl.BlockSpec((_BN, DK), lambda i: (i, 0)),
        out_shape=jax.ShapeDtypeStruct((N, DK), _f32),
    )(outp)


def kernel(x, edge_index, W_q, W_k, W_v, D):
    row = edge_index[0]
    col = edge_index[1]
    d = D.reshape(DK)
    q, k, v = _qkv(x, W_q, W_k, W_v)
    scores, tmax = _scores(k, q, row, col, d)
    ex, ssum2 = _expsum(scores, row, tmax)
    (outp,) = _scatter(ex, row, col, v, ssum2)
    return _final_add(outp)


# speedup vs baseline: 3.1241x; 3.1241x over previous
import functools
import math

import jax
import jax.numpy as jnp
from jax import lax
from jax.experimental import pallas as pl
from jax.experimental.pallas import tpu as pltpu
from jax.experimental.pallas import tpu_sc as plsc

N = 10000
E = 320000
F_IN = 128
DK = 128
SCALE = 1.0 / math.sqrt(DK)

NC = 2
NS = 16
NW = NC * NS
EPW = E // NW
CH = 80
NCHUNK = EPW // CH
NPAD = 10240
SLICE = NPAD // NS

_f32 = jnp.float32
_i32 = jnp.int32

_MESH = plsc.VectorSubcoreMesh(core_axis_name="c", subcore_axis_name="s")


def _qkv_body(x_ref, wq_ref, wk_ref, wv_ref, q_ref, k_ref, v_ref):
    xb = x_ref[...]
    dn = (((1,), (1,)), ((), ()))
    q_ref[...] = lax.dot_general(xb, wq_ref[...], dn, preferred_element_type=_f32)
    k_ref[...] = lax.dot_general(xb, wk_ref[...], dn, preferred_element_type=_f32)
    v_ref[...] = lax.dot_general(xb, wv_ref[...], dn, preferred_element_type=_f32)


_BN = 400


def _qkv(x, wq, wk, wv):
    grid = (N // _BN,)
    return pl.pallas_call(
        _qkv_body,
        grid=grid,
        in_specs=[
            pl.BlockSpec((_BN, F_IN), lambda i: (i, 0)),
            pl.BlockSpec((DK, F_IN), lambda i: (0, 0)),
            pl.BlockSpec((DK, F_IN), lambda i: (0, 0)),
            pl.BlockSpec((DK, F_IN), lambda i: (0, 0)),
        ],
        out_specs=[
            pl.BlockSpec((_BN, DK), lambda i: (i, 0)),
            pl.BlockSpec((_BN, DK), lambda i: (i, 0)),
            pl.BlockSpec((_BN, DK), lambda i: (i, 0)),
        ],
        out_shape=[
            jax.ShapeDtypeStruct((N, DK), _f32),
            jax.ShapeDtypeStruct((N, DK), _f32),
            jax.ShapeDtypeStruct((N, DK), _f32),
        ],
    )(x, wq, wk, wv)


def _scores_body(k_hbm, q_hbm, row_hbm, col_hbm, d_hbm,
                 scores_hbm, tmax_hbm,
                 idx_r, idx_c, krows, qrows, dvec, sbuf, mxbuf, sem):
    cid = lax.axis_index("c")
    sid = lax.axis_index("s")
    wid = sid * NC + cid
    base0 = wid * EPW
    pltpu.sync_copy(d_hbm, dvec)
    lanes = lax.iota(_i32, 16)

    def chunk_body(i, mrun):
        base = base0 + i * CH
        pltpu.sync_copy(row_hbm.at[pl.ds(base, CH)], idx_r)
        pltpu.sync_copy(col_hbm.at[pl.ds(base, CH)], idx_c)
        pltpu.async_copy(k_hbm.at[idx_r], krows, sem).wait()
        pltpu.async_copy(q_hbm.at[idx_c], qrows, sem).wait()

        def grp(g, mr):
            ev = g * 16 + lanes

            def d_body(dd, accs):
                acc_e, acc_s = accs
                dl = jnp.full((16,), dd, _i32)
                dv = plsc.load_gather(dvec, [dl])
                kv = plsc.load_gather(krows, [ev, dl])
                qv = plsc.load_gather(qrows, [ev, dl])
                diff = kv - qv
                return (acc_e + diff * diff * dv, acc_s + kv * qv * dv)

            acc_e, acc_s = lax.fori_loop(
                0, DK, d_body,
                (jnp.zeros((16,), _f32), jnp.zeros((16,), _f32)),
                unroll=2)
            rv = idx_r[pl.ds(g * 16, 16)]
            cv = idx_c[pl.ds(g * 16, 16)]
            sv = jnp.where(rv == cv, acc_s, acc_e) * SCALE
            sv = jnp.where(sv >= 0.0, sv, 0.2 * sv)
            sbuf[pl.ds(g * 16, 16)] = sv
            return jnp.maximum(mr, sv)

        mrun = lax.fori_loop(0, CH // 16, grp, mrun)
        pltpu.sync_copy(sbuf, scores_hbm.at[pl.ds(base, CH)])
        return mrun

    mrun = lax.fori_loop(0, NCHUNK, chunk_body,
                         jnp.full((16,), -jnp.inf, _f32))
    mxbuf[...] = mrun
    pltpu.sync_copy(mxbuf, tmax_hbm.at[wid])


def _scores(k, q, row, col, d):
    fn = functools.partial(
        pl.kernel,
        out_type=[
            jax.ShapeDtypeStruct((E,), _f32),
            jax.ShapeDtypeStruct((NW, 16), _f32),
        ],
        mesh=_MESH,
        compiler_params=pltpu.CompilerParams(needs_layout_passes=False),
        scratch_types=[
            pltpu.VMEM((CH,), _i32),
            pltpu.VMEM((CH,), _i32),
            pltpu.VMEM((CH, DK), _f32),
            pltpu.VMEM((CH, DK), _f32),
            pltpu.VMEM((DK,), _f32),
            pltpu.VMEM((CH,), _f32),
            pltpu.VMEM((16,), _f32),
            pltpu.SemaphoreType.DMA,
        ],
    )(_scores_body)
    return fn(k, q, row, col, d)


def _expsum_body(scores_hbm, row_hbm, tmax_hbm,
                 ex_hbm, ssum2_hbm,
                 sbuf, rbuf, exbuf, ssum_v, tmv, tmp, accb, spart, sem):
    cid = lax.axis_index("c")
    sid = lax.axis_index("s")
    wid = sid * NC + cid
    base0 = wid * EPW
    lanes = lax.iota(_i32, 16)
    pltpu.sync_copy(tmax_hbm, tmv)
    mv = jnp.full((16,), -jnp.inf, _f32)
    for i in range(NW):
        mv = jnp.maximum(mv, tmv[i, :])
    gmax = jnp.max(mv)

    def zb(i, _):
        ssum_v[pl.ds(i * 16, 16)] = jnp.zeros((16,), _f32)
        return 0

    lax.fori_loop(0, NPAD // 16, zb, 0)

    def chunk_body(i, _):
        base = base0 + i * CH
        pltpu.sync_copy(scores_hbm.at[pl.ds(base, CH)], sbuf)
        pltpu.sync_copy(row_hbm.at[pl.ds(base, CH)], rbuf)
        for g in range(CH // 16):
            sv = sbuf[pl.ds(g * 16, 16)]
            exbuf[pl.ds(g * 16, 16)] = jnp.exp(sv - gmax)
        pltpu.sync_copy(exbuf, ex_hbm.at[pl.ds(base, CH)])

        def eb(e, _):
            ezv = jnp.full((16,), e, _i32)
            rg = plsc.load_gather(rbuf, [ezv])
            r_s = rg[0]
            exg = plsc.load_gather(exbuf, [ezv])
            old = ssum_v[pl.ds(r_s, 16)]
            ssum_v[pl.ds(r_s, 16)] = old + jnp.where(lanes == 0, exg, 0.0)
            return 0

        lax.fori_loop(0, CH, eb, 0)
        return 0

    lax.fori_loop(0, NCHUNK, chunk_body, 0)

    pltpu.sync_copy(ssum_v, spart.at[sid])
    plsc.subcore_barrier()
    off = sid * SLICE
    pltpu.sync_copy(spart.at[0, pl.ds(off, SLICE)], accb)
    for p in range(1, NS):
        pltpu.sync_copy(spart.at[p, pl.ds(off, SLICE)], tmp)

        def ab(j, _):
            accb[pl.ds(j * 16, 16)] = (accb[pl.ds(j * 16, 16)]
                                       + tmp[pl.ds(j * 16, 16)])
            return 0

        lax.fori_loop(0, SLICE // 16, ab, 0)
    pltpu.sync_copy(accb, ssum2_hbm.at[cid, pl.ds(off, SLICE)])


def _expsum(scores, row, tmax):
    fn = functools.partial(
        pl.kernel,
        out_type=[
            jax.ShapeDtypeStruct((E,), _f32),
            jax.ShapeDtypeStruct((NC, NPAD), _f32),
        ],
        mesh=_MESH,
        compiler_params=pltpu.CompilerParams(needs_layout_passes=False),
        scratch_types=[
            pltpu.VMEM((CH,), _f32),
            pltpu.VMEM((CH,), _i32),
            pltpu.VMEM((CH,), _f32),
            pltpu.VMEM((NPAD,), _f32),
            pltpu.VMEM((NW, 16), _f32),
            pltpu.VMEM((SLICE,), _f32),
            pltpu.VMEM((SLICE,), _f32),
            pltpu.VMEM_SHARED((NS, NPAD), _f32),
            pltpu.SemaphoreType.DMA,
        ],
    )(_expsum_body)
    return fn(scores, row, tmax)


def _scatter_body(ex_hbm, row_hbm, col_hbm, v_hbm, ssum2_hbm,
                  outp_hbm,
                  idx_r, idx_c, exbuf, abuf, vrows, msg, ssum_v, s2buf,
                  out_acc, sem):
    cid = lax.axis_index("c")
    sid = lax.axis_index("s")
    wid = sid * NC + cid
    base0 = wid * EPW

    pltpu.sync_copy(ssum2_hbm.at[0], ssum_v)
    pltpu.sync_copy(ssum2_hbm.at[1], s2buf)

    def ad(j, _):
        ssum_v[pl.ds(j * 16, 16)] = (ssum_v[pl.ds(j * 16, 16)]
                                     + s2buf[pl.ds(j * 16, 16)])
        return 0

    lax.fori_loop(0, NPAD // 16, ad, 0)

    def zm(j, _):
        msg[j // (DK // 16), pl.ds((j % (DK // 16)) * 16, 16)] = jnp.zeros((16,), _f32)
        return 0

    lax.fori_loop(0, CH * DK // 16, zm, 0)
    for h in range(SLICE // CH):
        pltpu.sync_copy(msg, out_acc.at[pl.ds(sid * SLICE + h * CH, CH)])
    plsc.subcore_barrier()

    def chunk_body(i, _):
        base = base0 + i * CH
        pltpu.sync_copy(row_hbm.at[pl.ds(base, CH)], idx_r)
        pltpu.sync_copy(col_hbm.at[pl.ds(base, CH)], idx_c)
        pltpu.sync_copy(ex_hbm.at[pl.ds(base, CH)], exbuf)
        pltpu.async_copy(v_hbm.at[idx_c], vrows, sem).wait()
        for g in range(CH // 16):
            rv = idx_r[pl.ds(g * 16, 16)]
            sden = plsc.load_gather(ssum_v, [rv])
            abuf[pl.ds(g * 16, 16)] = (exbuf[pl.ds(g * 16, 16)]
                                       / (sden + 1e-16))

        def eb(e, _):
            av = plsc.load_gather(abuf, [jnp.full((16,), e, _i32)])
            for j in range(DK // 16):
                msg[e, pl.ds(j * 16, 16)] = vrows[e, pl.ds(j * 16, 16)] * av
            return 0

        lax.fori_loop(0, CH, eb, 0)
        pltpu.sync_copy(msg, out_acc.at[idx_r], add=True)
        return 0

    lax.fori_loop(0, NCHUNK, chunk_body, 0)
    plsc.subcore_barrier()
    pltpu.sync_copy(out_acc.at[pl.ds(sid * SLICE, SLICE)],
                    outp_hbm.at[cid, pl.ds(sid * SLICE, SLICE)])


def _scatter(ex, row, col, v, ssum2):
    fn = functools.partial(
        pl.kernel,
        out_type=[jax.ShapeDtypeStruct((NC, NPAD, DK), _f32)],
        mesh=_MESH,
        compiler_params=pltpu.CompilerParams(needs_layout_passes=False),
        scratch_types=[
            pltpu.VMEM((CH,), _i32),
            pltpu.VMEM((CH,), _i32),
            pltpu.VMEM((CH,), _f32),
            pltpu.VMEM((CH,), _f32),
            pltpu.VMEM((CH, DK), _f32),
            pltpu.VMEM((CH, DK), _f32),
            pltpu.VMEM((NPAD,), _f32),
            pltpu.VMEM((NPAD,), _f32),
            pltpu.VMEM_SHARED((NPAD, DK), _f32),
            pltpu.SemaphoreType.DMA,
        ],
    )(_scatter_body)
    return fn(ex, row, col, v, ssum2)


def _add_body(p_ref, o_ref):
    o_ref[...] = p_ref[0] + p_ref[1]


def _final_add(outp):
    grid = (N // _BN,)
    return pl.pallas_call(
        _add_body,
        grid=grid,
        in_specs=[pl.BlockSpec((NC, _BN, DK), lambda i: (0, i, 0))],
        out_specs=pl.BlockSpec((_BN, DK), lambda i: (i, 0)),
        out_shape=jax.ShapeDtypeStruct((N, DK), _f32),
    )(outp)


def kernel(x, edge_index, W_q, W_k, W_v, D):
    row = edge_index[0]
    col = edge_index[1]
    d = D.reshape(DK)
    q, k, v = _qkv(x, W_q, W_k, W_v)
    scores, tmax = _scores(k, q, row, col, d)
    ex, ssum2 = _expsum(scores, row, tmax)
    (outp,) = _scatter(ex, row, col, v, ssum2)
    return _final_add(outp)


# speedup vs baseline: 4.6018x; 1.4730x over previous
import functools
import math

import jax
import jax.numpy as jnp
from jax import lax
from jax.experimental import pallas as pl
from jax.experimental.pallas import tpu as pltpu
from jax.experimental.pallas import tpu_sc as plsc

N = 10000
E = 320000
F_IN = 128
DK = 128
SCALE = 1.0 / math.sqrt(DK)

NC = 2
NS = 16
NW = NC * NS
EPW = E // NW
CH = 80
NCHUNK = EPW // CH
NPAD = 10240
SLICE = NPAD // NS

_f32 = jnp.float32
_i32 = jnp.int32

_MESH = plsc.VectorSubcoreMesh(core_axis_name="c", subcore_axis_name="s")
_SC_PARAMS = pltpu.CompilerParams(needs_layout_passes=False)


def _qkv_body(x_ref, wq_ref, wk_ref, wv_ref, d_ref,
              q_ref, kd_ref, vlo_ref, vhi_ref):
    xb = x_ref[...]
    dn = (((1,), (1,)), ((), ()))
    q = lax.dot_general(xb, wq_ref[...], dn, preferred_element_type=_f32)
    k = lax.dot_general(xb, wk_ref[...], dn, preferred_element_type=_f32)
    v = lax.dot_general(xb, wv_ref[...], dn, preferred_element_type=_f32)
    q_ref[...] = q
    kd_ref[...] = k * d_ref[...]
    vlo_ref[...] = v[:, :DK // 2]
    vhi_ref[...] = v[:, DK // 2:]


_BN = 400


def _qkv(x, wq, wk, wv, d2):
    grid = (N // _BN,)
    return pl.pallas_call(
        _qkv_body,
        grid=grid,
        in_specs=[
            pl.BlockSpec((_BN, F_IN), lambda i: (i, 0)),
            pl.BlockSpec((DK, F_IN), lambda i: (0, 0)),
            pl.BlockSpec((DK, F_IN), lambda i: (0, 0)),
            pl.BlockSpec((DK, F_IN), lambda i: (0, 0)),
            pl.BlockSpec((1, DK), lambda i: (0, 0)),
        ],
        out_specs=[
            pl.BlockSpec((_BN, DK), lambda i: (i, 0)),
            pl.BlockSpec((_BN, DK), lambda i: (i, 0)),
            pl.BlockSpec((_BN, DK // 2), lambda i: (i, 0)),
            pl.BlockSpec((_BN, DK // 2), lambda i: (i, 0)),
        ],
        out_shape=[
            jax.ShapeDtypeStruct((N, DK), _f32),
            jax.ShapeDtypeStruct((N, DK), _f32),
            jax.ShapeDtypeStruct((N, DK // 2), _f32),
            jax.ShapeDtypeStruct((N, DK // 2), _f32),
        ],
    )(x, wq, wk, wv, d2)


def _scal_body(kd_ref, q_ref, d_ref, scal_ref):
    kd = kd_ref[...]
    q = q_ref[...]
    dv = d_ref[...]
    kk = jnp.sum(jnp.where(dv == 0.0, 0.0, kd * kd / dv), axis=1)
    qq = jnp.sum(q * q * dv, axis=1)
    scal_ref[...] = jnp.concatenate(
        [kk[None], qq[None], jnp.zeros((6, N), _f32)], axis=0)


def _scal(kd, q, d2):
    return pl.pallas_call(
        _scal_body,
        out_shape=jax.ShapeDtypeStruct((8, N), _f32),
    )(kd, q, d2)


def _scores_body(kd_hbm, q_hbm, row_hbm, col_hbm, scal_hbm,
                 scores_hbm, tmax_hbm,
                 rall, cally, kkv, qqv, sall, mxbuf,
                 bk0, bq0, bk1, bq1, semk0, semq0, semk1, semq1):
    cid = lax.axis_index("c")
    sid = lax.axis_index("s")
    wid = sid * NC + cid
    base0 = wid * EPW
    pltpu.sync_copy(row_hbm.at[pl.ds(base0, EPW)], rall)
    pltpu.sync_copy(col_hbm.at[pl.ds(base0, EPW)], cally)
    pltpu.sync_copy(scal_hbm.at[0], kkv)
    pltpu.sync_copy(scal_hbm.at[1], qqv)
    lanes = lax.iota(_i32, 16)

    def issue(c, bk, bq, sk, sq):
        off = c * CH
        pltpu.async_copy(kd_hbm.at[rall.at[pl.ds(off, CH)]], bk, sk)
        pltpu.async_copy(q_hbm.at[cally.at[pl.ds(off, CH)]], bq, sq)

    def wait(bk, bq, sk, sq):
        pltpu.make_async_copy(kd_hbm.at[rall.at[pl.ds(0, CH)]], bk, sk).wait()
        pltpu.make_async_copy(q_hbm.at[cally.at[pl.ds(0, CH)]], bq, sq).wait()

    def compute(c, bk, bq, mr):
        off = c * CH
        for g in range(CH // 16):
            ev = g * 16 + lanes

            def d_body(dd, acc):
                dl = jnp.full((16,), dd, _i32)
                kv = plsc.load_gather(bk, [ev, dl])
                qv = plsc.load_gather(bq, [ev, dl])
                return acc + kv * qv

            acc = lax.fori_loop(0, DK, d_body, jnp.zeros((16,), _f32),
                                unroll=4)
            rv = rall[pl.ds(off + g * 16, 16)]
            cv = cally[pl.ds(off + g * 16, 16)]
            kkg = plsc.load_gather(kkv, [rv])
            qqg = plsc.load_gather(qqv, [cv])
            se = kkg - 2.0 * acc + qqg
            sv = jnp.where(rv == cv, acc, se) * SCALE
            sv = jnp.where(sv >= 0.0, sv, 0.2 * sv)
            sall[pl.ds(off + g * 16, 16)] = sv
            mr = jnp.maximum(mr, sv)
        return mr

    issue(0, bk0, bq0, semk0, semq0)

    def body(i, mr):
        a = 2 * i + 1
        issue(a, bk1, bq1, semk1, semq1)
        wait(bk0, bq0, semk0, semq0)
        mr = compute(2 * i, bk0, bq0, mr)
        issue(2 * i + 2, bk0, bq0, semk0, semq0)
        wait(bk1, bq1, semk1, semq1)
        mr = compute(a, bk1, bq1, mr)
        return mr

    mr = lax.fori_loop(0, NCHUNK // 2, body,
                       jnp.full((16,), -jnp.inf, _f32))
    wait(bk0, bq0, semk0, semq0)
    mr = compute(NCHUNK - 1, bk0, bq0, mr)

    pltpu.sync_copy(sall, scores_hbm.at[pl.ds(base0, EPW)])
    mxbuf[...] = mr
    pltpu.sync_copy(mxbuf, tmax_hbm.at[wid])


def _scores(kd, q, row, col, scal):
    fn = functools.partial(
        pl.kernel,
        out_type=[
            jax.ShapeDtypeStruct((E,), _f32),
            jax.ShapeDtypeStruct((NW, 16), _f32),
        ],
        mesh=_MESH,
        compiler_params=_SC_PARAMS,
        scratch_types=[
            pltpu.VMEM((EPW,), _i32),
            pltpu.VMEM((EPW,), _i32),
            pltpu.VMEM((N,), _f32),
            pltpu.VMEM((N,), _f32),
            pltpu.VMEM((EPW,), _f32),
            pltpu.VMEM((16,), _f32),
            pltpu.VMEM((CH, DK), _f32),
            pltpu.VMEM((CH, DK), _f32),
            pltpu.VMEM((CH, DK), _f32),
            pltpu.VMEM((CH, DK), _f32),
            pltpu.SemaphoreType.DMA,
            pltpu.SemaphoreType.DMA,
            pltpu.SemaphoreType.DMA,
            pltpu.SemaphoreType.DMA,
        ],
    )(_scores_body)
    return fn(kd, q, row, col, scal)


def _expsum_body(scores_hbm, row_hbm, tmax_hbm,
                 ex_hbm, ssum2_hbm,
                 sall, rall, exall, ssum_v, tmv, tmp, accb, spart):
    cid = lax.axis_index("c")
    sid = lax.axis_index("s")
    wid = sid * NC + cid
    base0 = wid * EPW
    lanes = lax.iota(_i32, 16)
    pltpu.sync_copy(tmax_hbm, tmv)
    mv = jnp.full((16,), -jnp.inf, _f32)
    for i in range(NW):
        mv = jnp.maximum(mv, tmv[i, :])
    gmax = jnp.max(mv)

    def zb(i, _):
        ssum_v[pl.ds(i * 16, 16)] = jnp.zeros((16,), _f32)
        return 0

    lax.fori_loop(0, NPAD // 16, zb, 0)

    pltpu.sync_copy(scores_hbm.at[pl.ds(base0, EPW)], sall)
    pltpu.sync_copy(row_hbm.at[pl.ds(base0, EPW)], rall)

    def eg(g, _):
        sv = sall[pl.ds(g * 16, 16)]
        exall[pl.ds(g * 16, 16)] = jnp.exp(sv - gmax)
        return 0

    lax.fori_loop(0, EPW // 16, eg, 0)
    pltpu.sync_copy(exall, ex_hbm.at[pl.ds(base0, EPW)])

    def eb(e, _):
        ezv = jnp.full((16,), e, _i32)
        rg = plsc.load_gather(rall, [ezv])
        r_s = rg[0]
        exg = plsc.load_gather(exall, [ezv])
        old = ssum_v[pl.ds(r_s, 16)]
        ssum_v[pl.ds(r_s, 16)] = old + jnp.where(lanes == 0, exg, 0.0)
        return 0

    lax.fori_loop(0, EPW, eb, 0)

    pltpu.sync_copy(ssum_v, spart.at[sid])
    plsc.subcore_barrier()
    off = sid * SLICE
    pltpu.sync_copy(spart.at[0, pl.ds(off, SLICE)], accb)
    for p in range(1, NS):
        pltpu.sync_copy(spart.at[p, pl.ds(off, SLICE)], tmp)

        def ab(j, _):
            accb[pl.ds(j * 16, 16)] = (accb[pl.ds(j * 16, 16)]
                                       + tmp[pl.ds(j * 16, 16)])
            return 0

        lax.fori_loop(0, SLICE // 16, ab, 0)
    pltpu.sync_copy(accb, ssum2_hbm.at[cid, pl.ds(off, SLICE)])


def _expsum(scores, row, tmax):
    fn = functools.partial(
        pl.kernel,
        out_type=[
            jax.ShapeDtypeStruct((E,), _f32),
            jax.ShapeDtypeStruct((NC, NPAD), _f32),
        ],
        mesh=_MESH,
        compiler_params=_SC_PARAMS,
        scratch_types=[
            pltpu.VMEM((EPW,), _f32),
            pltpu.VMEM((EPW,), _i32),
            pltpu.VMEM((EPW,), _f32),
            pltpu.VMEM((NPAD,), _f32),
            pltpu.VMEM((NW, 16), _f32),
            pltpu.VMEM((SLICE,), _f32),
            pltpu.VMEM((SLICE,), _f32),
            pltpu.VMEM_SHARED((NS, NPAD), _f32),
        ],
    )(_expsum_body)
    return fn(scores, row, tmax)


_HD = DK // 2


def _scatter_body(ex_hbm, row3_hbm, col_hbm, vlo_hbm, vhi_hbm, ssum2_hbm,
                  outp_hbm,
                  exall, cally, ssum_v, s2buf, alpha, rall2,
                  bv0, bv1, m0, m1,
                  out_acc, semg0, semg1, sems0, sems1):
    cid = lax.axis_index("c")
    sid = lax.axis_index("s")
    wid = sid * NC + cid
    base0 = wid * EPW

    pltpu.sync_copy(ssum2_hbm.at[0], ssum_v)
    pltpu.sync_copy(ssum2_hbm.at[1], s2buf)

    def ad(j, _):
        ssum_v[pl.ds(j * 16, 16)] = (ssum_v[pl.ds(j * 16, 16)]
                                     + s2buf[pl.ds(j * 16, 16)])
        return 0

    lax.fori_loop(0, NPAD // 16, ad, 0)

    pltpu.sync_copy(ex_hbm.at[pl.ds(base0, EPW)], exall)
    pltpu.sync_copy(col_hbm.at[pl.ds(base0, EPW)], cally)
    pltpu.sync_copy(row3_hbm.at[wid], rall2)

    def al(c, _):
        for g in range(CH // 16):
            rv = rall2[c, pl.ds(g * 16, 16)]
            sden = plsc.load_gather(ssum_v, [rv])
            off = c * CH + g * 16
            alpha[pl.ds(off, 16)] = exall[pl.ds(off, 16)] / (sden + 1e-16)
        return 0

    lax.fori_loop(0, NCHUNK, al, 0)

    def zero_acc():
        def zm(j, _):
            m0[j // (_HD // 16), pl.ds((j % (_HD // 16)) * 16, 16)] = (
                jnp.zeros((16,), _f32))
            return 0

        lax.fori_loop(0, CH * _HD // 16, zm, 0)
        for hh in range(SLICE // CH):
            pltpu.sync_copy(m0, out_acc.at[pl.ds(sid * SLICE + hh * CH, CH)])

    for h in range(2):
        vh_hbm = vlo_hbm if h == 0 else vhi_hbm
        zero_acc()
        plsc.subcore_barrier()

        def issueg(c, bv, sg):
            pltpu.async_copy(vh_hbm.at[cally.at[pl.ds(c * CH, CH)]], bv, sg)

        def waitg(bv, sg):
            pltpu.make_async_copy(vh_hbm.at[cally.at[pl.ds(0, CH)]],
                                  bv, sg).wait()

        def compute(c, bv, msg):
            def eb(e, _):
                av = plsc.load_gather(alpha,
                                      [jnp.full((16,), c * CH, _i32) + e])
                for j in range(_HD // 16):
                    msg[e, pl.ds(j * 16, 16)] = bv[e, pl.ds(j * 16, 16)] * av
                return 0

            lax.fori_loop(0, CH, eb, 0)

        def issues(c, msg, ss):
            pltpu.async_copy(msg, out_acc.at[rall2.at[c]], ss, add=True)

        def waits(msg, ss):
            pltpu.make_async_copy(msg, out_acc.at[rall2.at[0]], ss).wait()

        issueg(0, bv0, semg0)

        def body(i, _):
            a = 2 * i + 1
            issueg(a, bv1, semg1)

            @pl.when(i > 0)
            def _():
                waits(m0, sems0)

            waitg(bv0, semg0)
            compute(2 * i, bv0, m0)
            issues(2 * i, m0, sems0)
            issueg(2 * i + 2, bv0, semg0)

            @pl.when(i > 0)
            def _():
                waits(m1, sems1)

            waitg(bv1, semg1)
            compute(a, bv1, m1)
            issues(a, m1, sems1)
            return 0

        lax.fori_loop(0, NCHUNK // 2, body, 0)
        waits(m0, sems0)
        waitg(bv0, semg0)
        compute(NCHUNK - 1, bv0, m0)
        issues(NCHUNK - 1, m0, sems0)
        waits(m1, sems1)
        waits(m0, sems0)
        plsc.subcore_barrier()
        pltpu.sync_copy(out_acc.at[pl.ds(sid * SLICE, SLICE)],
                        outp_hbm.at[cid, h, pl.ds(sid * SLICE, SLICE)])
        plsc.subcore_barrier()


def _scatter(ex, row3, col, vlo, vhi, ssum2):
    fn = functools.partial(
        pl.kernel,
        out_type=[jax.ShapeDtypeStruct((NC, 2, NPAD, _HD), _f32)],
        mesh=_MESH,
        compiler_params=pltpu.CompilerParams(needs_layout_passes=False,
                                             use_tc_tiling_on_sc=False),
        scratch_types=[
            pltpu.VMEM((EPW,), _f32),
            pltpu.VMEM((EPW,), _i32),
            pltpu.VMEM((NPAD,), _f32),
            pltpu.VMEM((NPAD,), _f32),
            pltpu.VMEM((EPW,), _f32),
            pltpu.VMEM((NCHUNK, CH), _i32),
            pltpu.VMEM((CH, _HD), _f32),
            pltpu.VMEM((CH, _HD), _f32),
            pltpu.VMEM((CH, _HD), _f32),
            pltpu.VMEM((CH, _HD), _f32),
            pltpu.VMEM_SHARED((NPAD, _HD), _f32),
            pltpu.SemaphoreType.DMA,
            pltpu.SemaphoreType.DMA,
            pltpu.SemaphoreType.DMA,
            pltpu.SemaphoreType.DMA,
        ],
    )(_scatter_body)
    return fn(ex, row3, col, vlo, vhi, ssum2)


def _add_body(p_ref, o_ref):
    lo = p_ref[0, 0] + p_ref[1, 0]
    hi = p_ref[0, 1] + p_ref[1, 1]
    o_ref[...] = jnp.concatenate([lo, hi], axis=1)


def _final_add(outp):
    grid = (N // _BN,)
    return pl.pallas_call(
        _add_body,
        grid=grid,
        in_specs=[pl.BlockSpec((NC, 2, _BN, _HD), lambda i: (0, 0, i, 0))],
        out_specs=pl.BlockSpec((_BN, DK), lambda i: (i, 0)),
        out_shape=jax.ShapeDtypeStruct((N, DK), _f32),
    )(outp)


def kernel(x, edge_index, W_q, W_k, W_v, D):
    row = edge_index[0]
    col = edge_index[1]
    d2 = D.reshape(1, DK)
    row3 = row.reshape(NW, NCHUNK, CH)
    q, kd, vlo, vhi = _qkv(x, W_q, W_k, W_v, d2)
    scal = _scal(kd, q, d2)
    scores, tmax = _scores(kd, q, row, col, scal)
    ex, ssum2 = _expsum(scores, row, tmax)
    (outp,) = _scatter(ex, row3, col, vlo, vhi, ssum2)
    return _final_add(outp)
